# baseline (speedup 1.0000x reference)
import functools

import jax
import jax.numpy as jnp
from jax import lax
from jax.experimental import pallas as pl
from jax.experimental.pallas import tpu as pltpu
from jax.experimental.pallas import tpu_sc as plsc

N = 10000
E = 160000
D = 256
H = 128
ODIM = 128
NLAYER = 4

NC, NS = 2, 16
EPT = E // NS
CH = 128
NCHUNK = 80
EPAD = NCHUNK * CH - EPT
NP = N + 8
CW = 16

RCH = 80
NRCH = N // RCH

_mesh = plsc.VectorSubcoreMesh(core_axis_name="c", subcore_axis_name="s",
                               num_cores=NC, num_subcores=NS)


def _tile_row_loop(s, fn):
    def body(j, carry):
        k = s + j * NS

        @pl.when(k < NRCH)
        def _():
            fn(pl.multiple_of(k * RCH, 8))

        return carry

    lax.fori_loop(0, (NRCH + NS - 1) // NS, body, 0)


NBUF = 2
IDEPTH = 4
NSUB = 4
HG = CH // NSUB


def _sc_agg_body(h_hbm, zrows, sd_hbm, agg_hbm,
                 agg_sp, iring, bufs, isems, gsems, ssems):
    c = lax.axis_index("c")
    s = lax.axis_index("s")
    _tile_row_loop(s, lambda r0: pltpu.sync_copy(
        zrows.at[pl.ds(r0, RCH)], agg_sp.at[pl.ds(r0, RCH)]))
    plsc.subcore_barrier()

    def fire_idx(g, sl):
        pltpu.async_copy(sd_hbm.at[c, s, g], iring[sl], isems[sl])

    def wait_idx(sl):
        pltpu.make_async_copy(sd_hbm.at[c, s, 0], iring[sl],
                              isems[sl]).wait()

    def fire_gath(sl, b):
        for u in range(NSUB):
            pltpu.async_copy(h_hbm.at[iring[sl].at[0, pl.ds(u * HG, HG)]],
                             bufs[b].at[pl.ds(u * HG, HG)], gsems[b])

    def wait_gath(sl, b):
        for u in range(NSUB):
            pltpu.make_async_copy(
                h_hbm.at[iring[sl].at[0, pl.ds(u * HG, HG)]],
                bufs[b].at[pl.ds(u * HG, HG)], gsems[b]).wait()

    def fire_scat(sl, b):
        pltpu.async_copy(bufs[b], agg_sp.at[iring[sl].at[1]], ssems[b],
                         add=True)

    def wait_scat(b):
        pltpu.make_async_copy(bufs[b], agg_sp.at[iring[0].at[1]],
                              ssems[b]).wait()

    for sl in range(IDEPTH):
        fire_idx(sl, sl)

    def outer(go, carry):
        for j in range(IDEPTH):
            g = IDEPTH * go + j
            b = j & 1

            @pl.when(g >= 2)
            def _():
                wait_scat(b)

            @pl.when(jnp.logical_and(g >= 2, g + 2 < NCHUNK))
            def _():
                fire_idx(g + 2, (j + 2) % IDEPTH)

            wait_idx(j)
            fire_gath(j, b)

            @pl.when(g >= 1)
            def _():
                wait_gath((j + 3) % IDEPTH, 1 - b)
                fire_scat((j + 3) % IDEPTH, 1 - b)

        return carry

    lax.fori_loop(0, NCHUNK // IDEPTH, outer, 0)
    wait_gath(IDEPTH - 1, 1)
    fire_scat(IDEPTH - 1, 1)
    wait_scat(0)
    wait_scat(1)
    plsc.subcore_barrier()
    obase = c * N
    _tile_row_loop(s, lambda r0: pltpu.sync_copy(
        agg_sp.at[pl.ds(r0, RCH)], agg_hbm.at[pl.ds(obase + r0, RCH)]))


_CNT_SPLIT = NCHUNK // NC


def _sc_cnt_body(dst_hbm, zrows, ones_hbm, cnt_hbm,
                 cnt_sp, dst_t, ones_v, ssems):
    c = lax.axis_index("c")
    s = lax.axis_index("s")
    _tile_row_loop(s, lambda r0: pltpu.sync_copy(
        zrows.at[pl.ds(r0, RCH)], cnt_sp.at[pl.ds(r0, RCH)]))
    pltpu.sync_copy(dst_hbm.at[s], dst_t)
    pltpu.sync_copy(ones_hbm, ones_v)
    plsc.subcore_barrier()

    cbase = c * _CNT_SPLIT

    for b in range(NBUF):
        pltpu.async_copy(ones_v, cnt_sp.at[dst_t.at[cbase + b]], ssems[b],
                         add=True)

    def outer(g2, carry):
        for b in range(NBUF):
            i = cbase + g2 * NBUF + b
            pltpu.make_async_copy(ones_v, cnt_sp.at[dst_t.at[i]],
                                  ssems[b]).wait()

            @pl.when(i + NBUF < cbase + _CNT_SPLIT)
            def _():
                pltpu.async_copy(ones_v, cnt_sp.at[dst_t.at[i + NBUF]],
                                 ssems[b], add=True)

        return carry

    lax.fori_loop(0, _CNT_SPLIT // NBUF, outer, 0)
    plsc.subcore_barrier()
    obase = c * N
    _tile_row_loop(s, lambda r0: pltpu.sync_copy(
        cnt_sp.at[pl.ds(r0, RCH)], cnt_hbm.at[pl.ds(obase + r0, RCH)]))


_agg_call = pl.kernel(
    _sc_agg_body,
    out_type=jax.ShapeDtypeStruct((NC * N, H), jnp.float32),
    mesh=_mesh,
    scratch_types=[
        pltpu.VMEM_SHARED((NP, H), jnp.float32),
        [pltpu.VMEM((2, CH), jnp.int32)] * IDEPTH,
        [pltpu.VMEM((CH, H), jnp.float32)] * NBUF,
        [pltpu.SemaphoreType.DMA] * IDEPTH,
        [pltpu.SemaphoreType.DMA] * NBUF,
        [pltpu.SemaphoreType.DMA] * NBUF,
    ],
)

_cnt_call = pl.kernel(
    _sc_cnt_body,
    out_type=jax.ShapeDtypeStruct((NC * N, H), jnp.float32),
    mesh=_mesh,
    scratch_types=[
        pltpu.VMEM_SHARED((NP, H), jnp.float32),
        pltpu.VMEM((NCHUNK, CH), jnp.int32),
        pltpu.VMEM((CH, H), jnp.float32),
        [pltpu.SemaphoreType.DMA] * NBUF,
    ],
)


R = 2000
GRID = N // R


def _tc_proj_body(x_ref, w_ref, b_ref, h_ref):
    h = jnp.dot(x_ref[...], w_ref[...], preferred_element_type=jnp.float32)
    h = jnp.maximum(h + b_ref[...], 0.0)
    h_ref[...] = jnp.stack([h[:, :H], h[:, H:]], axis=0)


def _tc_pre_body(x_ref, wr_ref, xr_ref):
    xr_ref[...] = jnp.dot(x_ref[...], wr_ref[...],
                          preferred_element_type=jnp.float32)


def _tc_layer_body(xr_ref, a_ref, c_ref, wl_ref, bl_ref,
                   g_ref, be_ref, wp_ref, bp_ref,
                   out_ref, h_ref, *, has_next):
    cnt = c_ref[0, :, 0:1] + c_ref[1, :, 0:1]
    inv = 1.0 / jnp.maximum(cnt, 1.0)
    agg = jnp.concatenate([a_ref[0], a_ref[1]], axis=1)
    mean = agg * inv
    o = (jnp.dot(mean, wl_ref[...], preferred_element_type=jnp.float32)
         + bl_ref[...] + xr_ref[...])
    nrm = jnp.sqrt(jnp.sum(o * o, axis=-1, keepdims=True))
    o = o / jnp.maximum(nrm, 1e-12)
    mu = jnp.mean(o, axis=-1, keepdims=True)
    var = jnp.mean((o - mu) * (o - mu), axis=-1, keepdims=True)
    o = (o - mu) / jnp.sqrt(var + 1e-5)
    o = o * g_ref[...] + be_ref[...]
    o = jnp.maximum(o, 0.0)
    out_ref[...] = o
    if has_next:
        hn = jnp.dot(o, wp_ref[...], preferred_element_type=jnp.float32)
        hn = jnp.maximum(hn + bp_ref[...], 0.0)
        h_ref[...] = jnp.stack([hn[:, :H], hn[:, H:]], axis=0)


def _tc_cls_body(o0_ref, o1_ref, o2_ref, o3_ref, w_ref, wc_ref, bc_ref,
                 out_ref):
    fused = (o0_ref[...] * w_ref[0]
             + o1_ref[...] * w_ref[1]
             + o2_ref[...] * w_ref[2]
             + o3_ref[...] * w_ref[3])
    out_ref[...] = (jnp.dot(fused, wc_ref[...],
                            preferred_element_type=jnp.float32) + bc_ref[...])


def _rows(i):
    return (i, 0)


def _full(i):
    return (0, 0)


def _rows3(i):
    return (0, i, 0)


_bs_x = pl.BlockSpec((R, D), _rows)
_bs_h3 = pl.BlockSpec((NC, R, H), _rows3)
_bs_c3 = pl.BlockSpec((NC, R, H), _rows3)
_bs_w = pl.BlockSpec((D, D), _full)
_bs_b = pl.BlockSpec((1, D), _full)
_bs_wc = pl.BlockSpec((D, ODIM), _full)
_bs_bc = pl.BlockSpec((1, ODIM), _full)
_bs_smem = pl.BlockSpec(memory_space=pltpu.SMEM)

_proj_call = pl.pallas_call(
    _tc_proj_body,
    grid=(GRID,),
    in_specs=[_bs_x, _bs_w, _bs_b],
    out_specs=_bs_h3,
    out_shape=jax.ShapeDtypeStruct((NC, N, H), jnp.float32),
)

_pre_call = pl.pallas_call(
    _tc_pre_body,
    grid=(GRID,),
    in_specs=[_bs_x, _bs_w],
    out_specs=_bs_x,
    out_shape=jax.ShapeDtypeStruct((N, D), jnp.float32),
)

_layer_specs = [_bs_x, _bs_h3, _bs_c3, _bs_w, _bs_b,
                _bs_b, _bs_b, _bs_w, _bs_b]

_layer_call_mid = pl.pallas_call(
    functools.partial(_tc_layer_body, has_next=True),
    grid=(GRID,),
    in_specs=_layer_specs,
    out_specs=(_bs_x, _bs_h3),
    out_shape=(jax.ShapeDtypeStruct((N, D), jnp.float32),
               jax.ShapeDtypeStruct((NC, N, H), jnp.float32)),
)

_layer_call_last = pl.pallas_call(
    functools.partial(_tc_layer_body, has_next=False),
    grid=(GRID,),
    in_specs=_layer_specs,
    out_specs=(_bs_x, _bs_h3),
    out_shape=(jax.ShapeDtypeStruct((N, D), jnp.float32),
               jax.ShapeDtypeStruct((NC, N, H), jnp.float32)),
)

_cls_call = pl.pallas_call(
    _tc_cls_body,
    grid=(GRID,),
    in_specs=[_bs_x, _bs_x, _bs_x, _bs_x, _bs_smem, _bs_wc, _bs_bc],
    out_specs=pl.BlockSpec((R, ODIM), _rows),
    out_shape=jax.ShapeDtypeStruct((N, ODIM), jnp.float32),
)


def kernel(node_features, edge_index, params):
    src = edge_index[0]
    dst = edge_index[1]
    zrows = jnp.zeros((N, H), jnp.float32)
    ones_ch = jnp.ones((CH, H), jnp.float32)
    w_fus = jax.nn.softmax(params['fusion'])
    pad_s = jnp.zeros((NS, EPAD), jnp.int32)
    pad_d = jnp.full((NS, EPAD), N, jnp.int32)
    srcp = jnp.concatenate([src.reshape(NS, EPT), pad_s], axis=1)
    dstp = jnp.concatenate([dst.reshape(NS, EPT), pad_d], axis=1)
    src2 = jnp.stack([srcp, srcp + N]).reshape(NC, NS, NCHUNK, CH)
    dst3 = dstp.reshape(NS, NCHUNK, CH)
    sd = jnp.stack(
        [src2, jnp.broadcast_to(dst3, (NC, NS, NCHUNK, CH))], axis=3)

    cnt2 = _cnt_call(dst3, zrows, ones_ch).reshape(NC, N, H)

    x = node_features
    outs = []
    b = lambda v: v.reshape(1, -1)
    h3 = _proj_call(x, params['proj_W_0'], b(params['proj_b_0']))
    for l in range(NLAYER):
        agg2 = _agg_call(h3.reshape(NC * N, H), zrows, sd)
        xr = _pre_call(x, params['lin_r_W_%d' % l])
        agg2 = agg2.reshape(NC, N, H)
        has_next = l < NLAYER - 1
        call = _layer_call_mid if has_next else _layer_call_last
        nl = l + 1 if has_next else l
        out, h3 = call(
            xr, agg2, cnt2,
            params['lin_l_W_%d' % l], b(params['lin_l_b_%d' % l]),
            b(params['ln_g_%d' % l]), b(params['ln_b_%d' % l]),
            params['proj_W_%d' % nl], b(params['proj_b_%d' % nl]))
        outs.append(out)
        x = out

    return _cls_call(outs[0], outs[1], outs[2], outs[3], w_fus,
                     params['cls_W'], b(params['cls_b']))


# speedup vs baseline: 1.0095x; 1.0095x over previous
import functools

import jax
import jax.numpy as jnp
from jax import lax
from jax.experimental import pallas as pl
from jax.experimental.pallas import tpu as pltpu
from jax.experimental.pallas import tpu_sc as plsc

N = 10000
E = 160000
D = 256
H = 128
ODIM = 128
NLAYER = 4

NC, NS = 2, 16
EPT = E // NS
CH = 128
NCHUNK = 80
EPAD = NCHUNK * CH - EPT
NP = N + 8
CW = 16

RCH = 80
NRCH = N // RCH

_mesh = plsc.VectorSubcoreMesh(core_axis_name="c", subcore_axis_name="s",
                               num_cores=NC, num_subcores=NS)


def _tile_row_loop(s, fn):
    def body(j, carry):
        k = s + j * NS

        @pl.when(k < NRCH)
        def _():
            fn(pl.multiple_of(k * RCH, 8))

        return carry

    lax.fori_loop(0, (NRCH + NS - 1) // NS, body, 0)


NBUF = 2
IDEPTH = 4
NSUB = 4
HG = CH // NSUB


def _sc_agg_body(h_hbm, zrows, sd_hbm, agg_hbm,
                 agg_sp, iring, bufs, isems, gsems, ssems):
    c = lax.axis_index("c")
    s = lax.axis_index("s")

    def fire_idx(g, sl):
        pltpu.async_copy(sd_hbm.at[c, s, g], iring[sl], isems[sl])

    def wait_idx(sl):
        pltpu.make_async_copy(sd_hbm.at[c, s, 0], iring[sl],
                              isems[sl]).wait()

    def fire_gath(sl, b):
        for u in range(NSUB):
            pltpu.async_copy(h_hbm.at[iring[sl].at[0, pl.ds(u * HG, HG)]],
                             bufs[b].at[pl.ds(u * HG, HG)], gsems[b])

    def wait_gath(sl, b):
        for u in range(NSUB):
            pltpu.make_async_copy(
                h_hbm.at[iring[sl].at[0, pl.ds(u * HG, HG)]],
                bufs[b].at[pl.ds(u * HG, HG)], gsems[b]).wait()

    def fire_scat(sl, b):
        pltpu.async_copy(bufs[b], agg_sp.at[iring[sl].at[1]], ssems[b],
                         add=True)

    def wait_scat(b):
        pltpu.make_async_copy(bufs[b], agg_sp.at[iring[0].at[1]],
                              ssems[b]).wait()

    for sl in range(IDEPTH):
        fire_idx(sl, sl)
    _tile_row_loop(s, lambda r0: pltpu.sync_copy(
        zrows.at[pl.ds(r0, RCH)], agg_sp.at[pl.ds(r0, RCH)]))
    plsc.subcore_barrier()

    def outer(go, carry):
        for j in range(IDEPTH):
            g = IDEPTH * go + j
            b = j & 1

            @pl.when(g >= 2)
            def _():
                wait_scat(b)

            @pl.when(jnp.logical_and(g >= 2, g + 2 < NCHUNK))
            def _():
                fire_idx(g + 2, (j + 2) % IDEPTH)

            wait_idx(j)
            fire_gath(j, b)

            @pl.when(g >= 1)
            def _():
                wait_gath((j + 3) % IDEPTH, 1 - b)
                fire_scat((j + 3) % IDEPTH, 1 - b)

        return carry

    lax.fori_loop(0, NCHUNK // IDEPTH, outer, 0)
    wait_gath(IDEPTH - 1, 1)
    fire_scat(IDEPTH - 1, 1)
    wait_scat(0)
    wait_scat(1)
    plsc.subcore_barrier()
    obase = c * N
    _tile_row_loop(s, lambda r0: pltpu.sync_copy(
        agg_sp.at[pl.ds(r0, RCH)], agg_hbm.at[pl.ds(obase + r0, RCH)]))


_CNT_SPLIT = NCHUNK // NC


def _sc_cnt_body(dst_hbm, zrows, ones_hbm, cnt_hbm,
                 cnt_sp, dst_t, ones_v, ssems):
    c = lax.axis_index("c")
    s = lax.axis_index("s")
    _tile_row_loop(s, lambda r0: pltpu.sync_copy(
        zrows.at[pl.ds(r0, RCH)], cnt_sp.at[pl.ds(r0, RCH)]))
    pltpu.sync_copy(dst_hbm.at[s], dst_t)
    pltpu.sync_copy(ones_hbm, ones_v)
    plsc.subcore_barrier()

    cbase = c * _CNT_SPLIT

    for b in range(NBUF):
        pltpu.async_copy(ones_v, cnt_sp.at[dst_t.at[cbase + b]], ssems[b],
                         add=True)

    def outer(g2, carry):
        for b in range(NBUF):
            i = cbase + g2 * NBUF + b
            pltpu.make_async_copy(ones_v, cnt_sp.at[dst_t.at[i]],
                                  ssems[b]).wait()

            @pl.when(i + NBUF < cbase + _CNT_SPLIT)
            def _():
                pltpu.async_copy(ones_v, cnt_sp.at[dst_t.at[i + NBUF]],
                                 ssems[b], add=True)

        return carry

    lax.fori_loop(0, _CNT_SPLIT // NBUF, outer, 0)
    plsc.subcore_barrier()
    obase = c * N
    _tile_row_loop(s, lambda r0: pltpu.sync_copy(
        cnt_sp.at[pl.ds(r0, RCH)], cnt_hbm.at[pl.ds(obase + r0, RCH)]))


_agg_call = pl.kernel(
    _sc_agg_body,
    out_type=jax.ShapeDtypeStruct((NC * N, H), jnp.float32),
    mesh=_mesh,
    scratch_types=[
        pltpu.VMEM_SHARED((NP, H), jnp.float32),
        [pltpu.VMEM((2, CH), jnp.int32)] * IDEPTH,
        [pltpu.VMEM((CH, H), jnp.float32)] * NBUF,
        [pltpu.SemaphoreType.DMA] * IDEPTH,
        [pltpu.SemaphoreType.DMA] * NBUF,
        [pltpu.SemaphoreType.DMA] * NBUF,
    ],
)

_cnt_call = pl.kernel(
    _sc_cnt_body,
    out_type=jax.ShapeDtypeStruct((NC * N, H), jnp.float32),
    mesh=_mesh,
    scratch_types=[
        pltpu.VMEM_SHARED((NP, H), jnp.float32),
        pltpu.VMEM((NCHUNK, CH), jnp.int32),
        pltpu.VMEM((CH, H), jnp.float32),
        [pltpu.SemaphoreType.DMA] * NBUF,
    ],
)


R = 2000
GRID = N // R


def _tc_proj_body(x_ref, w_ref, b_ref, h_ref):
    h = jnp.dot(x_ref[...], w_ref[...], preferred_element_type=jnp.float32)
    h = jnp.maximum(h + b_ref[...], 0.0)
    h_ref[...] = jnp.stack([h[:, :H], h[:, H:]], axis=0)


def _tc_layer_body(x_ref, a_ref, c_ref, wl_ref, bl_ref,
                   wr_ref, g_ref, be_ref, wp_ref, bp_ref,
                   out_ref, h_ref, *, has_next):
    cnt = c_ref[0, :, 0:1] + c_ref[1, :, 0:1]
    inv = 1.0 / jnp.maximum(cnt, 1.0)
    agg = jnp.concatenate([a_ref[0], a_ref[1]], axis=1)
    mean = agg * inv
    o = (jnp.dot(mean, wl_ref[...], preferred_element_type=jnp.float32)
         + bl_ref[...]
         + jnp.dot(x_ref[...], wr_ref[...], preferred_element_type=jnp.float32))
    nrm = jnp.sqrt(jnp.sum(o * o, axis=-1, keepdims=True))
    o = o / jnp.maximum(nrm, 1e-12)
    mu = jnp.mean(o, axis=-1, keepdims=True)
    var = jnp.mean((o - mu) * (o - mu), axis=-1, keepdims=True)
    o = (o - mu) / jnp.sqrt(var + 1e-5)
    o = o * g_ref[...] + be_ref[...]
    o = jnp.maximum(o, 0.0)
    out_ref[...] = o
    if has_next:
        hn = jnp.dot(o, wp_ref[...], preferred_element_type=jnp.float32)
        hn = jnp.maximum(hn + bp_ref[...], 0.0)
        h_ref[...] = jnp.stack([hn[:, :H], hn[:, H:]], axis=0)


def _tc_cls_body(o0_ref, o1_ref, o2_ref, o3_ref, w_ref, wc_ref, bc_ref,
                 out_ref):
    fused = (o0_ref[...] * w_ref[0]
             + o1_ref[...] * w_ref[1]
             + o2_ref[...] * w_ref[2]
             + o3_ref[...] * w_ref[3])
    out_ref[...] = (jnp.dot(fused, wc_ref[...],
                            preferred_element_type=jnp.float32) + bc_ref[...])


def _rows(i):
    return (i, 0)


def _full(i):
    return (0, 0)


def _rows3(i):
    return (0, i, 0)


_bs_x = pl.BlockSpec((R, D), _rows)
_bs_h3 = pl.BlockSpec((NC, R, H), _rows3)
_bs_c3 = pl.BlockSpec((NC, R, H), _rows3)
_bs_w = pl.BlockSpec((D, D), _full)
_bs_b = pl.BlockSpec((1, D), _full)
_bs_wc = pl.BlockSpec((D, ODIM), _full)
_bs_bc = pl.BlockSpec((1, ODIM), _full)
_bs_smem = pl.BlockSpec(memory_space=pltpu.SMEM)

_proj_call = pl.pallas_call(
    _tc_proj_body,
    grid=(GRID,),
    in_specs=[_bs_x, _bs_w, _bs_b],
    out_specs=_bs_h3,
    out_shape=jax.ShapeDtypeStruct((NC, N, H), jnp.float32),
)

_layer_specs = [_bs_x, _bs_h3, _bs_c3, _bs_w, _bs_b, _bs_w,
                _bs_b, _bs_b, _bs_w, _bs_b]

_layer_call_mid = pl.pallas_call(
    functools.partial(_tc_layer_body, has_next=True),
    grid=(GRID,),
    in_specs=_layer_specs,
    out_specs=(_bs_x, _bs_h3),
    out_shape=(jax.ShapeDtypeStruct((N, D), jnp.float32),
               jax.ShapeDtypeStruct((NC, N, H), jnp.float32)),
)

_layer_call_last = pl.pallas_call(
    functools.partial(_tc_layer_body, has_next=False),
    grid=(GRID,),
    in_specs=_layer_specs,
    out_specs=(_bs_x, _bs_h3),
    out_shape=(jax.ShapeDtypeStruct((N, D), jnp.float32),
               jax.ShapeDtypeStruct((NC, N, H), jnp.float32)),
)

_cls_call = pl.pallas_call(
    _tc_cls_body,
    grid=(GRID,),
    in_specs=[_bs_x, _bs_x, _bs_x, _bs_x, _bs_smem, _bs_wc, _bs_bc],
    out_specs=pl.BlockSpec((R, ODIM), _rows),
    out_shape=jax.ShapeDtypeStruct((N, ODIM), jnp.float32),
)


def kernel(node_features, edge_index, params):
    src = edge_index[0]
    dst = edge_index[1]
    zrows = jnp.zeros((N, H), jnp.float32)
    ones_ch = jnp.ones((CH, H), jnp.float32)
    w_fus = jax.nn.softmax(params['fusion'])
    pad_s = jnp.zeros((NS, EPAD), jnp.int32)
    pad_d = jnp.full((NS, EPAD), N, jnp.int32)
    srcp = jnp.concatenate([src.reshape(NS, EPT), pad_s], axis=1)
    dstp = jnp.concatenate([dst.reshape(NS, EPT), pad_d], axis=1)
    src2 = jnp.stack([srcp, srcp + N]).reshape(NC, NS, NCHUNK, CH)
    dst3 = dstp.reshape(NS, NCHUNK, CH)
    sd = jnp.stack(
        [src2, jnp.broadcast_to(dst3, (NC, NS, NCHUNK, CH))], axis=3)

    cnt2 = _cnt_call(dst3, zrows, ones_ch).reshape(NC, N, H)

    x = node_features
    outs = []
    b = lambda v: v.reshape(1, -1)
    h3 = _proj_call(x, params['proj_W_0'], b(params['proj_b_0']))
    for l in range(NLAYER):
        agg2 = _agg_call(h3.reshape(NC * N, H), zrows, sd)
        agg2 = agg2.reshape(NC, N, H)
        has_next = l < NLAYER - 1
        call = _layer_call_mid if has_next else _layer_call_last
        nl = l + 1 if has_next else l
        out, h3 = call(
            x, agg2, cnt2,
            params['lin_l_W_%d' % l], b(params['lin_l_b_%d' % l]),
            params['lin_r_W_%d' % l],
            b(params['ln_g_%d' % l]), b(params['ln_b_%d' % l]),
            params['proj_W_%d' % nl], b(params['proj_b_%d' % nl]))
        outs.append(out)
        x = out

    return _cls_call(outs[0], outs[1], outs[2], outs[3], w_fus,
                     params['cls_W'], b(params['cls_b']))


# speedup vs baseline: 1.0217x; 1.0120x over previous
import functools

import jax
import jax.numpy as jnp
from jax import lax
from jax.experimental import pallas as pl
from jax.experimental.pallas import tpu as pltpu
from jax.experimental.pallas import tpu_sc as plsc

N = 10000
E = 160000
D = 256
H = 128
ODIM = 128
NLAYER = 4

NC, NS = 2, 16
EPT = E // NS
CH = 128
NCHUNK = 80
EPAD = NCHUNK * CH - EPT
NP = N + 8
CW = 16

RCH = 80
NRCH = N // RCH

_mesh = plsc.VectorSubcoreMesh(core_axis_name="c", subcore_axis_name="s",
                               num_cores=NC, num_subcores=NS)


def _tile_row_loop(s, fn):
    def body(j, carry):
        k = s + j * NS

        @pl.when(k < NRCH)
        def _():
            fn(pl.multiple_of(k * RCH, 8))

        return carry

    lax.fori_loop(0, (NRCH + NS - 1) // NS, body, 0)


NBUF = 2
IDEPTH = 4
NSUB = 4
HG = CH // NSUB


def _sc_agg_body(h_hbm, zrows, sd_hbm, agg_hbm,
                 agg_sp, iring, bufs, isems, gsems, ssems):
    c = lax.axis_index("c")
    s = lax.axis_index("s")

    def fire_idx(g, sl):
        pltpu.async_copy(sd_hbm.at[c, s, g], iring[sl], isems[sl])

    def wait_idx(sl):
        pltpu.make_async_copy(sd_hbm.at[c, s, 0], iring[sl],
                              isems[sl]).wait()

    def fire_gath(sl, b):
        for u in range(NSUB):
            pltpu.async_copy(h_hbm.at[iring[sl].at[0, pl.ds(u * HG, HG)]],
                             bufs[b].at[pl.ds(u * HG, HG)], gsems[b])

    def wait_gath(sl, b):
        for u in range(NSUB):
            pltpu.make_async_copy(
                h_hbm.at[iring[sl].at[0, pl.ds(u * HG, HG)]],
                bufs[b].at[pl.ds(u * HG, HG)], gsems[b]).wait()

    def fire_scat(sl, b):
        pltpu.async_copy(bufs[b], agg_sp.at[iring[sl].at[1]], ssems[b],
                         add=True)

    def wait_scat(b):
        pltpu.make_async_copy(bufs[b], agg_sp.at[iring[0].at[1]],
                              ssems[b]).wait()

    for sl in range(IDEPTH):
        fire_idx(sl, sl)
    _tile_row_loop(s, lambda r0: pltpu.sync_copy(
        zrows.at[pl.ds(r0, RCH)], agg_sp.at[pl.ds(r0, RCH)]))
    plsc.subcore_barrier()

    def outer(go, carry):
        for j in range(IDEPTH):
            g = IDEPTH * go + j
            b = j & 1

            @pl.when(g >= 2)
            def _():
                wait_scat(b)

            @pl.when(jnp.logical_and(g >= 2, g + 2 < NCHUNK))
            def _():
                fire_idx(g + 2, (j + 2) % IDEPTH)

            wait_idx(j)
            fire_gath(j, b)

            @pl.when(g >= 1)
            def _():
                wait_gath((j + 3) % IDEPTH, 1 - b)
                fire_scat((j + 3) % IDEPTH, 1 - b)

        return carry

    lax.fori_loop(0, NCHUNK // IDEPTH, outer, 0)
    wait_gath(IDEPTH - 1, 1)
    fire_scat(IDEPTH - 1, 1)
    wait_scat(0)
    wait_scat(1)
    plsc.subcore_barrier()
    obase = c * N
    _tile_row_loop(s, lambda r0: pltpu.sync_copy(
        agg_sp.at[pl.ds(r0, RCH)], agg_hbm.at[pl.ds(obase + r0, RCH)]))


_CNT_SPLIT = NCHUNK // NC


def _sc_cnt_body(dst_hbm, zrows, ones_hbm, cnt_hbm,
                 cnt_sp, dst_t, ones_v, ssems):
    c = lax.axis_index("c")
    s = lax.axis_index("s")
    _tile_row_loop(s, lambda r0: pltpu.sync_copy(
        zrows.at[pl.ds(r0, RCH)], cnt_sp.at[pl.ds(r0, RCH)]))
    pltpu.sync_copy(dst_hbm.at[s], dst_t)
    pltpu.sync_copy(ones_hbm, ones_v)
    plsc.subcore_barrier()

    cbase = c * _CNT_SPLIT

    for b in range(NBUF):
        pltpu.async_copy(ones_v, cnt_sp.at[dst_t.at[cbase + b]], ssems[b],
                         add=True)

    def outer(g2, carry):
        for b in range(NBUF):
            i = cbase + g2 * NBUF + b
            pltpu.make_async_copy(ones_v, cnt_sp.at[dst_t.at[i]],
                                  ssems[b]).wait()

            @pl.when(i + NBUF < cbase + _CNT_SPLIT)
            def _():
                pltpu.async_copy(ones_v, cnt_sp.at[dst_t.at[i + NBUF]],
                                 ssems[b], add=True)

        return carry

    lax.fori_loop(0, _CNT_SPLIT // NBUF, outer, 0)
    plsc.subcore_barrier()
    obase = c * N
    _tile_row_loop(s, lambda r0: pltpu.sync_copy(
        cnt_sp.at[pl.ds(r0, RCH)], cnt_hbm.at[pl.ds(obase + r0, RCH)]))


_agg_call = pl.kernel(
    _sc_agg_body,
    out_type=jax.ShapeDtypeStruct((NC * N, H), jnp.float32),
    mesh=_mesh,
    scratch_types=[
        pltpu.VMEM_SHARED((NP, H), jnp.float32),
        [pltpu.VMEM((2, CH), jnp.int32)] * IDEPTH,
        [pltpu.VMEM((CH, H), jnp.float32)] * NBUF,
        [pltpu.SemaphoreType.DMA] * IDEPTH,
        [pltpu.SemaphoreType.DMA] * NBUF,
        [pltpu.SemaphoreType.DMA] * NBUF,
    ],
)

_cnt_call = pl.kernel(
    _sc_cnt_body,
    out_type=jax.ShapeDtypeStruct((NC * N, H), jnp.float32),
    mesh=_mesh,
    scratch_types=[
        pltpu.VMEM_SHARED((NP, H), jnp.float32),
        pltpu.VMEM((NCHUNK, CH), jnp.int32),
        pltpu.VMEM((CH, H), jnp.float32),
        [pltpu.SemaphoreType.DMA] * NBUF,
    ],
)


R = 2000
GRID = N // R


def _tc_proj_body(x_ref, w_ref, b_ref, h_ref):
    h = jnp.dot(x_ref[...], w_ref[...], preferred_element_type=jnp.float32)
    h = jnp.maximum(h + b_ref[...], 0.0)
    h_ref[...] = jnp.stack([h[:, :H], h[:, H:]], axis=0)


def _tc_layer_body(x_ref, a_ref, c_ref, wl_ref, bl_ref,
                   wr_ref, g_ref, be_ref, wp_ref, bp_ref,
                   out_ref, h_ref, *, has_next):
    cnt = c_ref[0, :, 0:1] + c_ref[1, :, 0:1]
    inv = 1.0 / jnp.maximum(cnt, 1.0)
    agg = jnp.concatenate([a_ref[0], a_ref[1]], axis=1)
    mean = agg * inv
    o = (jnp.dot(mean, wl_ref[...], preferred_element_type=jnp.float32)
         + bl_ref[...]
         + jnp.dot(x_ref[...], wr_ref[...], preferred_element_type=jnp.float32))
    nrm = jnp.sqrt(jnp.sum(o * o, axis=-1, keepdims=True))
    o = o / jnp.maximum(nrm, 1e-12)
    mu = jnp.mean(o, axis=-1, keepdims=True)
    var = jnp.mean((o - mu) * (o - mu), axis=-1, keepdims=True)
    o = (o - mu) / jnp.sqrt(var + 1e-5)
    o = o * g_ref[...] + be_ref[...]
    o = jnp.maximum(o, 0.0)
    out_ref[...] = o
    if has_next:
        hn = jnp.dot(o, wp_ref[...], preferred_element_type=jnp.float32)
        hn = jnp.maximum(hn + bp_ref[...], 0.0)
        h_ref[...] = jnp.stack([hn[:, :H], hn[:, H:]], axis=0)


def _tc_last_body(x_ref, a_ref, c_ref, wl_ref, bl_ref, wr_ref, g_ref, be_ref,
                  o0_ref, o1_ref, o2_ref, w_ref, wc_ref, bc_ref, out_ref):
    cnt = c_ref[0, :, 0:1] + c_ref[1, :, 0:1]
    inv = 1.0 / jnp.maximum(cnt, 1.0)
    agg = jnp.concatenate([a_ref[0], a_ref[1]], axis=1)
    mean = agg * inv
    o = (jnp.dot(mean, wl_ref[...], preferred_element_type=jnp.float32)
         + bl_ref[...]
         + jnp.dot(x_ref[...], wr_ref[...], preferred_element_type=jnp.float32))
    nrm = jnp.sqrt(jnp.sum(o * o, axis=-1, keepdims=True))
    o = o / jnp.maximum(nrm, 1e-12)
    mu = jnp.mean(o, axis=-1, keepdims=True)
    var = jnp.mean((o - mu) * (o - mu), axis=-1, keepdims=True)
    o = (o - mu) / jnp.sqrt(var + 1e-5)
    o = o * g_ref[...] + be_ref[...]
    o = jnp.maximum(o, 0.0)
    fused = (o0_ref[...] * w_ref[0]
             + o1_ref[...] * w_ref[1]
             + o2_ref[...] * w_ref[2]
             + o * w_ref[3])
    out_ref[...] = (jnp.dot(fused, wc_ref[...],
                            preferred_element_type=jnp.float32) + bc_ref[...])


def _rows(i):
    return (i, 0)


def _full(i):
    return (0, 0)


def _rows3(i):
    return (0, i, 0)


_bs_x = pl.BlockSpec((R, D), _rows)
_bs_h3 = pl.BlockSpec((NC, R, H), _rows3)
_bs_c3 = pl.BlockSpec((NC, R, H), _rows3)
_bs_w = pl.BlockSpec((D, D), _full)
_bs_b = pl.BlockSpec((1, D), _full)
_bs_wc = pl.BlockSpec((D, ODIM), _full)
_bs_bc = pl.BlockSpec((1, ODIM), _full)
_bs_smem = pl.BlockSpec(memory_space=pltpu.SMEM)

_proj_call = pl.pallas_call(
    _tc_proj_body,
    grid=(GRID,),
    in_specs=[_bs_x, _bs_w, _bs_b],
    out_specs=_bs_h3,
    out_shape=jax.ShapeDtypeStruct((NC, N, H), jnp.float32),
)

_layer_specs = [_bs_x, _bs_h3, _bs_c3, _bs_w, _bs_b, _bs_w,
                _bs_b, _bs_b, _bs_w, _bs_b]

_layer_call_mid = pl.pallas_call(
    functools.partial(_tc_layer_body, has_next=True),
    grid=(GRID,),
    in_specs=_layer_specs,
    out_specs=(_bs_x, _bs_h3),
    out_shape=(jax.ShapeDtypeStruct((N, D), jnp.float32),
               jax.ShapeDtypeStruct((NC, N, H), jnp.float32)),
)

_last_call = pl.pallas_call(
    _tc_last_body,
    grid=(GRID,),
    in_specs=[_bs_x, _bs_h3, _bs_c3, _bs_w, _bs_b, _bs_w, _bs_b, _bs_b,
              _bs_x, _bs_x, _bs_x, _bs_smem, _bs_wc, _bs_bc],
    out_specs=pl.BlockSpec((R, ODIM), _rows),
    out_shape=jax.ShapeDtypeStruct((N, ODIM), jnp.float32),
)


def kernel(node_features, edge_index, params):
    src = edge_index[0]
    dst = edge_index[1]
    zrows = jnp.zeros((N, H), jnp.float32)
    ones_ch = jnp.ones((CH, H), jnp.float32)
    w_fus = jax.nn.softmax(params['fusion'])
    pad_s = jnp.zeros((NS, EPAD), jnp.int32)
    pad_d = jnp.full((NS, EPAD), N, jnp.int32)
    srcp = jnp.concatenate([src.reshape(NS, EPT), pad_s], axis=1)
    dstp = jnp.concatenate([dst.reshape(NS, EPT), pad_d], axis=1)
    src2 = jnp.stack([srcp, srcp + N]).reshape(NC, NS, NCHUNK, CH)
    dst3 = dstp.reshape(NS, NCHUNK, CH)
    sd = jnp.stack(
        [src2, jnp.broadcast_to(dst3, (NC, NS, NCHUNK, CH))], axis=3)

    cnt2 = _cnt_call(dst3, zrows, ones_ch).reshape(NC, N, H)

    x = node_features
    outs = []
    b = lambda v: v.reshape(1, -1)
    h3 = _proj_call(x, params['proj_W_0'], b(params['proj_b_0']))
    for l in range(NLAYER - 1):
        agg2 = _agg_call(h3.reshape(NC * N, H), zrows, sd)
        agg2 = agg2.reshape(NC, N, H)
        out, h3 = _layer_call_mid(
            x, agg2, cnt2,
            params['lin_l_W_%d' % l], b(params['lin_l_b_%d' % l]),
            params['lin_r_W_%d' % l],
            b(params['ln_g_%d' % l]), b(params['ln_b_%d' % l]),
            params['proj_W_%d' % (l + 1)], b(params['proj_b_%d' % (l + 1)]))
        outs.append(out)
        x = out

    agg2 = _agg_call(h3.reshape(NC * N, H), zrows, sd).reshape(NC, N, H)
    return _last_call(
        x, agg2, cnt2,
        params['lin_l_W_3'], b(params['lin_l_b_3']), params['lin_r_W_3'],
        b(params['ln_g_3']), b(params['ln_b_3']),
        outs[0], outs[1], outs[2], w_fus,
        params['cls_W'], b(params['cls_b']))


# speedup vs baseline: 1.0223x; 1.0007x over previous
import functools

import jax
import jax.numpy as jnp
from jax import lax
from jax.experimental import pallas as pl
from jax.experimental.pallas import tpu as pltpu
from jax.experimental.pallas import tpu_sc as plsc

N = 10000
E = 160000
D = 256
H = 128
ODIM = 128
NLAYER = 4

NC, NS = 2, 16
EPT = E // NS
CH = 128
NCHUNK = 80
EPAD = NCHUNK * CH - EPT
NP = N + 8

RCH = 80
NRCH = N // RCH

_mesh = plsc.VectorSubcoreMesh(core_axis_name="c", subcore_axis_name="s",
                               num_cores=NC, num_subcores=NS)


def _tile_row_loop(s, fn):
    def body(j, carry):
        k = s + j * NS

        @pl.when(k < NRCH)
        def _():
            fn(pl.multiple_of(k * RCH, 8))

        return carry

    lax.fori_loop(0, (NRCH + NS - 1) // NS, body, 0)


NBUF = 2
IDEPTH = 4
NSUB = 4
HG = CH // NSUB


def _sc_agg_body(h_hbm, zrows, sd_hbm, agg_hbm,
                 agg_sp, iring, bufs, isems, gsems, ssems):
    c = lax.axis_index("c")
    s = lax.axis_index("s")

    def fire_idx(g, sl):
        pltpu.async_copy(sd_hbm.at[c, s, g], iring[sl], isems[sl])

    def wait_idx(sl):
        pltpu.make_async_copy(sd_hbm.at[c, s, 0], iring[sl],
                              isems[sl]).wait()

    def fire_gath(sl, b):
        for u in range(NSUB):
            pltpu.async_copy(h_hbm.at[iring[sl].at[0, pl.ds(u * HG, HG)]],
                             bufs[b].at[pl.ds(u * HG, HG)], gsems[b])

    def wait_gath(sl, b):
        for u in range(NSUB):
            pltpu.make_async_copy(
                h_hbm.at[iring[sl].at[0, pl.ds(u * HG, HG)]],
                bufs[b].at[pl.ds(u * HG, HG)], gsems[b]).wait()

    def fire_scat(sl, b):
        pltpu.async_copy(bufs[b], agg_sp.at[iring[sl].at[1]], ssems[b],
                         add=True)

    def wait_scat(b):
        pltpu.make_async_copy(bufs[b], agg_sp.at[iring[0].at[1]],
                              ssems[b]).wait()

    for sl in range(IDEPTH):
        fire_idx(sl, sl)
    _tile_row_loop(s, lambda r0: pltpu.sync_copy(
        zrows.at[pl.ds(r0, RCH)], agg_sp.at[pl.ds(r0, RCH)]))
    plsc.subcore_barrier()

    def outer(go, carry):
        for j in range(IDEPTH):
            g = IDEPTH * go + j
            b = j & 1

            @pl.when(g >= 2)
            def _():
                wait_scat(b)

            @pl.when(jnp.logical_and(g >= 2, g + 2 < NCHUNK))
            def _():
                fire_idx(g + 2, (j + 2) % IDEPTH)

            wait_idx(j)
            fire_gath(j, b)

            @pl.when(g >= 1)
            def _():
                wait_gath((j + 3) % IDEPTH, 1 - b)
                fire_scat((j + 3) % IDEPTH, 1 - b)

        return carry

    lax.fori_loop(0, NCHUNK // IDEPTH, outer, 0)
    wait_gath(IDEPTH - 1, 1)
    fire_scat(IDEPTH - 1, 1)
    wait_scat(0)
    wait_scat(1)
    plsc.subcore_barrier()
    obase = c * N
    _tile_row_loop(s, lambda r0: pltpu.sync_copy(
        agg_sp.at[pl.ds(r0, RCH)], agg_hbm.at[pl.ds(obase + r0, RCH)]))


_CNT_SPLIT = NCHUNK // NC


def _sc_cnt_body(dst_hbm, zrows, ones_hbm, cnt_hbm,
                 cnt_sp, dst_t, ones_v, ssems):
    c = lax.axis_index("c")
    s = lax.axis_index("s")
    _tile_row_loop(s, lambda r0: pltpu.sync_copy(
        zrows.at[pl.ds(r0, RCH)], cnt_sp.at[pl.ds(r0, RCH)]))
    pltpu.sync_copy(dst_hbm.at[s], dst_t)
    pltpu.sync_copy(ones_hbm, ones_v)
    plsc.subcore_barrier()

    cbase = c * _CNT_SPLIT

    for b in range(NBUF):
        pltpu.async_copy(ones_v, cnt_sp.at[dst_t.at[cbase + b]], ssems[b],
                         add=True)

    def outer(g2, carry):
        for b in range(NBUF):
            i = cbase + g2 * NBUF + b
            pltpu.make_async_copy(ones_v, cnt_sp.at[dst_t.at[i]],
                                  ssems[b]).wait()

            @pl.when(i + NBUF < cbase + _CNT_SPLIT)
            def _():
                pltpu.async_copy(ones_v, cnt_sp.at[dst_t.at[i + NBUF]],
                                 ssems[b], add=True)

        return carry

    lax.fori_loop(0, _CNT_SPLIT // NBUF, outer, 0)
    plsc.subcore_barrier()
    obase = c * N
    _tile_row_loop(s, lambda r0: pltpu.sync_copy(
        cnt_sp.at[pl.ds(r0, RCH)], cnt_hbm.at[pl.ds(obase + r0, RCH)]))


_agg_call = pl.kernel(
    _sc_agg_body,
    out_type=jax.ShapeDtypeStruct((NC * N, H), jnp.float32),
    mesh=_mesh,
    scratch_types=[
        pltpu.VMEM_SHARED((NP, H), jnp.float32),
        [pltpu.VMEM((2, CH), jnp.int32)] * IDEPTH,
        [pltpu.VMEM((CH, H), jnp.float32)] * NBUF,
        [pltpu.SemaphoreType.DMA] * IDEPTH,
        [pltpu.SemaphoreType.DMA] * NBUF,
        [pltpu.SemaphoreType.DMA] * NBUF,
    ],
)

_cnt_call = pl.kernel(
    _sc_cnt_body,
    out_type=jax.ShapeDtypeStruct((NC * N, H), jnp.float32),
    mesh=_mesh,
    scratch_types=[
        pltpu.VMEM_SHARED((NP, H), jnp.float32),
        pltpu.VMEM((NCHUNK, CH), jnp.int32),
        pltpu.VMEM((CH, H), jnp.float32),
        [pltpu.SemaphoreType.DMA] * NBUF,
    ],
)


R = 2000
GRID = N // R


def _tc_proj_body(x_ref, w_ref, b_ref, h_ref):
    h = jnp.dot(x_ref[...], w_ref[...], preferred_element_type=jnp.float32)
    h = jnp.maximum(h + b_ref[...], 0.0)
    h_ref[...] = jnp.stack([h[:, :H], h[:, H:]], axis=0)


def _tc_layer_body(x_ref, a_ref, c_ref, wl_ref, bl_ref,
                   wr_ref, g_ref, be_ref, wp_ref, bp_ref,
                   out_ref, h_ref, *, has_next):
    cnt = c_ref[0, :, 0:1] + c_ref[1, :, 0:1]
    inv = 1.0 / jnp.maximum(cnt, 1.0)
    agg = jnp.concatenate([a_ref[0], a_ref[1]], axis=1)
    mean = agg * inv
    o = (jnp.dot(mean, wl_ref[...], preferred_element_type=jnp.float32)
         + bl_ref[...]
         + jnp.dot(x_ref[...], wr_ref[...], preferred_element_type=jnp.float32))
    nrm = jnp.sqrt(jnp.sum(o * o, axis=-1, keepdims=True))
    o = o / jnp.maximum(nrm, 1e-12)
    mu = jnp.mean(o, axis=-1, keepdims=True)
    var = jnp.mean((o - mu) * (o - mu), axis=-1, keepdims=True)
    o = (o - mu) / jnp.sqrt(var + 1e-5)
    o = o * g_ref[...] + be_ref[...]
    o = jnp.maximum(o, 0.0)
    out_ref[...] = o
    if has_next:
        hn = jnp.dot(o, wp_ref[...], preferred_element_type=jnp.float32)
        hn = jnp.maximum(hn + bp_ref[...], 0.0)
        h_ref[...] = jnp.stack([hn[:, :H], hn[:, H:]], axis=0)


def _tc_last_body(x_ref, a_ref, c_ref, wl_ref, bl_ref, wr_ref, g_ref, be_ref,
                  o0_ref, o1_ref, o2_ref, w_ref, wc_ref, bc_ref, out_ref):
    cnt = c_ref[0, :, 0:1] + c_ref[1, :, 0:1]
    inv = 1.0 / jnp.maximum(cnt, 1.0)
    agg = jnp.concatenate([a_ref[0], a_ref[1]], axis=1)
    mean = agg * inv
    o = (jnp.dot(mean, wl_ref[...], preferred_element_type=jnp.float32)
         + bl_ref[...]
         + jnp.dot(x_ref[...], wr_ref[...], preferred_element_type=jnp.float32))
    nrm = jnp.sqrt(jnp.sum(o * o, axis=-1, keepdims=True))
    o = o / jnp.maximum(nrm, 1e-12)
    mu = jnp.mean(o, axis=-1, keepdims=True)
    var = jnp.mean((o - mu) * (o - mu), axis=-1, keepdims=True)
    o = (o - mu) / jnp.sqrt(var + 1e-5)
    o = o * g_ref[...] + be_ref[...]
    o = jnp.maximum(o, 0.0)
    fused = (o0_ref[...] * w_ref[0]
             + o1_ref[...] * w_ref[1]
             + o2_ref[...] * w_ref[2]
             + o * w_ref[3])
    out_ref[...] = (jnp.dot(fused, wc_ref[...],
                            preferred_element_type=jnp.float32) + bc_ref[...])


def _rows(i):
    return (i, 0)


def _full(i):
    return (0, 0)


def _rows3(i):
    return (0, i, 0)


_bs_x = pl.BlockSpec((R, D), _rows)
_bs_h3 = pl.BlockSpec((NC, R, H), _rows3)
_bs_c3 = pl.BlockSpec((NC, R, H), _rows3)
_bs_w = pl.BlockSpec((D, D), _full)
_bs_b = pl.BlockSpec((1, D), _full)
_bs_wc = pl.BlockSpec((D, ODIM), _full)
_bs_bc = pl.BlockSpec((1, ODIM), _full)
_bs_smem = pl.BlockSpec(memory_space=pltpu.SMEM)

_proj_call = pl.pallas_call(
    _tc_proj_body,
    grid=(GRID,),
    in_specs=[_bs_x, _bs_w, _bs_b],
    out_specs=_bs_h3,
    out_shape=jax.ShapeDtypeStruct((NC, N, H), jnp.float32),
)

_layer_specs = [_bs_x, _bs_h3, _bs_c3, _bs_w, _bs_b, _bs_w,
                _bs_b, _bs_b, _bs_w, _bs_b]

_layer_call_mid = pl.pallas_call(
    functools.partial(_tc_layer_body, has_next=True),
    grid=(GRID,),
    in_specs=_layer_specs,
    out_specs=(_bs_x, _bs_h3),
    out_shape=(jax.ShapeDtypeStruct((N, D), jnp.float32),
               jax.ShapeDtypeStruct((NC, N, H), jnp.float32)),
)

_last_call = pl.pallas_call(
    _tc_last_body,
    grid=(GRID,),
    in_specs=[_bs_x, _bs_h3, _bs_c3, _bs_w, _bs_b, _bs_w, _bs_b, _bs_b,
              _bs_x, _bs_x, _bs_x, _bs_smem, _bs_wc, _bs_bc],
    out_specs=pl.BlockSpec((R, ODIM), _rows),
    out_shape=jax.ShapeDtypeStruct((N, ODIM), jnp.float32),
)


def kernel(node_features, edge_index, params):
    src = edge_index[0]
    dst = edge_index[1]
    zrows = jnp.zeros((N, H), jnp.float32)
    ones_ch = jnp.ones((CH, H), jnp.float32)
    w_fus = jax.nn.softmax(params['fusion'])
    pad_s = jnp.zeros((NS, EPAD), jnp.int32)
    pad_d = jnp.full((NS, EPAD), N, jnp.int32)
    srcp = jnp.concatenate([src.reshape(NS, EPT), pad_s], axis=1)
    dstp = jnp.concatenate([dst.reshape(NS, EPT), pad_d], axis=1)
    src2 = jnp.stack([srcp, srcp + N]).reshape(NC, NS, NCHUNK, CH)
    dst3 = dstp.reshape(NS, NCHUNK, CH)
    sd = jnp.stack(
        [src2, jnp.broadcast_to(dst3, (NC, NS, NCHUNK, CH))], axis=3)

    cnt2 = _cnt_call(dst3, zrows, ones_ch).reshape(NC, N, H)

    x = node_features
    outs = []
    b = lambda v: v.reshape(1, -1)
    h3 = _proj_call(x, params['proj_W_0'], b(params['proj_b_0']))
    for l in range(NLAYER - 1):
        agg2 = _agg_call(h3.reshape(NC * N, H), zrows, sd)
        agg2 = agg2.reshape(NC, N, H)
        out, h3 = _layer_call_mid(
            x, agg2, cnt2,
            params['lin_l_W_%d' % l], b(params['lin_l_b_%d' % l]),
            params['lin_r_W_%d' % l],
            b(params['ln_g_%d' % l]), b(params['ln_b_%d' % l]),
            params['proj_W_%d' % (l + 1)], b(params['proj_b_%d' % (l + 1)]))
        outs.append(out)
        x = out

    agg2 = _agg_call(h3.reshape(NC * N, H), zrows, sd).reshape(NC, N, H)
    return _last_call(
        x, agg2, cnt2,
        params['lin_l_W_3'], b(params['lin_l_b_3']), params['lin_r_W_3'],
        b(params['ln_g_3']), b(params['ln_b_3']),
        outs[0], outs[1], outs[2], w_fus,
        params['cls_W'], b(params['cls_b']))
